# baseline (speedup 1.0000x reference)
import functools

import jax
import jax.numpy as jnp
from jax import lax
from jax.experimental import pallas as pl
from jax.experimental.pallas import tpu as pltpu
from jax.experimental.pallas import tpu_sc as plsc


def _sc_add(B, S, D):
    NC, NS = 2, 16
    NW = NC * NS
    SW = S // NW
    C = 16
    n_chunks = SW // C
    n_steps = n_chunks * B
    NBUF = 4
    LOOKAHEAD = 2

    mesh = plsc.VectorSubcoreMesh(core_axis_name="c", subcore_axis_name="s")

    @functools.partial(
        pl.kernel,
        mesh=mesh,
        out_type=jax.ShapeDtypeStruct((B * S, D), jnp.float32),
        scratch_types=[
            pltpu.VMEM((2, C, D), jnp.float32),
            pltpu.VMEM((NBUF, C, D), jnp.float32),
            pltpu.SemaphoreType.DMA,
            pltpu.SemaphoreType.DMA,
            pltpu.SemaphoreType.DMA,
        ],
    )
    def run(x_hbm, pos_hbm, out_hbm, p_v, x_v, sem_in, sem_pos, sem_out):
        wid = lax.axis_index("s") * NC + lax.axis_index("c")
        s_base = wid * SW

        def row0(t):
            c, b = t // B, t % B
            return b * S + s_base + c * C

        def start_in(t):
            pltpu.async_copy(x_hbm.at[pl.ds(row0(t), C)], x_v.at[t % NBUF], sem_in)

        def start_pos(c):
            pltpu.async_copy(
                pos_hbm.at[pl.ds(s_base + c * C, C)], p_v.at[c % 2], sem_pos
            )

        def wait(src, dst, sem):
            pltpu.make_async_copy(src, dst, sem).wait()

        start_pos(0)
        for t in range(LOOKAHEAD):
            start_in(t)
        outs_waited = 0
        for t in range(n_steps):
            c = t // B
            if t % B == 0 and c + 1 < n_chunks:
                start_pos(c + 1)
            if t % B == 0:
                wait(pos_hbm.at[pl.ds(0, C)], p_v.at[c % 2], sem_pos)
            wait(x_hbm.at[pl.ds(0, C)], x_v.at[t % NBUF], sem_in)
            if t + LOOKAHEAD < n_steps:
                if t + LOOKAHEAD - NBUF >= 0:
                    wait(x_v.at[0], out_hbm.at[pl.ds(0, C)], sem_out)
                    outs_waited += 1
                start_in(t + LOOKAHEAD)

            xb = x_v.at[t % NBUF]
            pb = p_v.at[c % 2]

            def add_body(i, acc):
                r = i // 8
                j = (i % 8) * 128
                vals = [pb[r, pl.ds(j + k * 16, 16)] for k in range(8)]
                for k in range(8):
                    plsc.addupdate(xb.at[r, pl.ds(j + k * 16, 16)], vals[k])
                return acc

            lax.fori_loop(0, C * 8, add_body, 0)

            pltpu.async_copy(xb, out_hbm.at[pl.ds(row0(t), C)], sem_out)
        for _ in range(n_steps - outs_waited):
            wait(x_v.at[0], out_hbm.at[pl.ds(0, C)], sem_out)

    return run


def _tc_add_body(x_ref, p_ref, o_ref):
    o_ref[...] = x_ref[...] + p_ref[...][None, :, :]


def _tc_add(B, S, D, dtype):
    BS = 512
    return pl.pallas_call(
        _tc_add_body,
        grid=(S // BS,),
        in_specs=[
            pl.BlockSpec((B, BS, D), lambda j: (0, j, 0)),
            pl.BlockSpec((BS, D), lambda j: (j, 0)),
        ],
        out_specs=pl.BlockSpec((B, BS, D), lambda j: (0, j, 0)),
        out_shape=jax.ShapeDtypeStruct((B, S, D), dtype),
    )


def kernel(x, pos_weight):
    B, S, D = x.shape
    B_SC = B // 2
    pos = pos_weight[:S]
    sc_out = _sc_add(B_SC, S, D)(x[B - B_SC:].reshape(B_SC * S, D), pos)
    tc_out = _tc_add(B - B_SC, S, D, x.dtype)(x[: B - B_SC], pos)
    return jnp.concatenate([tc_out, sc_out.reshape(B_SC, S, D)], axis=0)


# speedup vs baseline: 2.2828x; 2.2828x over previous
import functools

import jax
import jax.numpy as jnp
from jax import lax
from jax.experimental import pallas as pl
from jax.experimental.pallas import tpu as pltpu
from jax.experimental.pallas import tpu_sc as plsc


def _sc_add(B, S, D):
    NC, NS = 2, 16
    NW = NC * NS
    SW = S // NW
    C = 16
    n_chunks = SW // C
    n_steps = n_chunks * B
    NBUF = 5
    LOOKAHEAD = 3

    mesh = plsc.VectorSubcoreMesh(core_axis_name="c", subcore_axis_name="s")

    @functools.partial(
        pl.kernel,
        mesh=mesh,
        out_type=jax.ShapeDtypeStruct((B * S, D), jnp.float32),
        scratch_types=[
            pltpu.VMEM((2, C, D), jnp.float32),
            pltpu.VMEM((NBUF, C, D), jnp.float32),
            pltpu.SemaphoreType.DMA,
            pltpu.SemaphoreType.DMA,
            pltpu.SemaphoreType.DMA,
        ],
    )
    def run(x_hbm, pos_hbm, out_hbm, p_v, x_v, sem_in, sem_pos, sem_out):
        wid = lax.axis_index("s") * NC + lax.axis_index("c")
        s_base = wid * SW

        def row0(t):
            c, b = t // B, t % B
            return b * S + s_base + c * C

        def start_in(t):
            pltpu.async_copy(x_hbm.at[pl.ds(row0(t), C)], x_v.at[t % NBUF], sem_in)

        def start_pos(c):
            pltpu.async_copy(
                pos_hbm.at[pl.ds(s_base + c * C, C)], p_v.at[c % 2], sem_pos
            )

        def wait(src, dst, sem):
            pltpu.make_async_copy(src, dst, sem).wait()

        start_pos(0)
        for t in range(LOOKAHEAD):
            start_in(t)
        outs_waited = 0
        for t in range(n_steps):
            c = t // B
            if t % B == 0 and c + 1 < n_chunks:
                start_pos(c + 1)
            if t % B == 0:
                wait(pos_hbm.at[pl.ds(0, C)], p_v.at[c % 2], sem_pos)
            wait(x_hbm.at[pl.ds(0, C)], x_v.at[t % NBUF], sem_in)
            if t + LOOKAHEAD < n_steps:
                if t + LOOKAHEAD - NBUF >= 0:
                    wait(x_v.at[0], out_hbm.at[pl.ds(0, C)], sem_out)
                    outs_waited += 1
                start_in(t + LOOKAHEAD)

            xb = x_v.at[t % NBUF]
            pb = p_v.at[c % 2]

            def add_body(i, acc):
                r = i // 8
                j = (i % 8) * 128
                vals = [pb[r, pl.ds(j + k * 16, 16)] for k in range(8)]
                for k in range(8):
                    plsc.addupdate(xb.at[r, pl.ds(j + k * 16, 16)], vals[k])
                return acc

            lax.fori_loop(0, C * 8, add_body, 0)

            pltpu.async_copy(xb, out_hbm.at[pl.ds(row0(t), C)], sem_out)
        for _ in range(n_steps - outs_waited):
            wait(x_v.at[0], out_hbm.at[pl.ds(0, C)], sem_out)

    return run


def kernel(x, pos_weight):
    B, S, D = x.shape
    out = _sc_add(B, S, D)(x.reshape(B * S, D), pos_weight[:S])
    return out.reshape(B, S, D)
